# baseline (speedup 1.0000x reference)
import jax
import jax.numpy as jnp
from jax import lax
from jax.experimental import pallas as pl
from jax.experimental.pallas import tpu as pltpu
from jax.experimental.pallas import tpu_sc as plsc

B = 16384
D = 64
L = 16
NC = 2
NS = 16
NW = NC * NS
BPW = B // NW
NB = BPW // L


def _bias_mf_body(u_hbm, i_hbm, ub_hbm, ib_hbm, usr_hbm, itm_hbm, out_hbm,
                  uidx, iidx, urows, irows, ubv, ibv, outv, sem):
  wid = lax.axis_index("s") * NC + lax.axis_index("c")
  base = wid * BPW

  pltpu.sync_copy(usr_hbm.at[pl.ds(base, BPW)], uidx)
  pltpu.sync_copy(itm_hbm.at[pl.ds(base, BPW)], iidx)

  cu = pltpu.async_copy(u_hbm.at[uidx], urows, sem)
  ci = pltpu.async_copy(i_hbm.at[iidx], irows, sem)
  cub = pltpu.async_copy(ub_hbm.at[uidx], ubv, sem)
  cib = pltpu.async_copy(ib_hbm.at[iidx], ibv, sem)
  cu.wait()
  ci.wait()
  cub.wait()
  cib.wait()

  lane = lax.iota(jnp.int32, L)

  def batch(t, carry):
    rows = t * L + lane
    acc = ubv[pl.ds(t * L, L)] + ibv[pl.ds(t * L, L)]
    for d in range(D):
      col = jnp.full((L,), d, jnp.int32)
      uv = plsc.load_gather(urows, [rows, col])
      iv = plsc.load_gather(irows, [rows, col])
      acc = acc + uv * iv
    outv[pl.ds(t * L, L)] = acc
    return carry

  lax.fori_loop(0, NB, batch, 0)
  pltpu.sync_copy(outv, out_hbm.at[pl.ds(base, BPW)])


def kernel(uEmbeds, iEmbeds, uBias, iBias, usr, itm):
  f = pl.kernel(
      _bias_mf_body,
      out_type=jax.ShapeDtypeStruct((B,), jnp.float32),
      mesh=plsc.VectorSubcoreMesh(core_axis_name="c", subcore_axis_name="s"),
      compiler_params=pltpu.CompilerParams(
          needs_layout_passes=False, use_tc_tiling_on_sc=False),
      scratch_types=[
          pltpu.VMEM((BPW,), jnp.int32),
          pltpu.VMEM((BPW,), jnp.int32),
          pltpu.VMEM((BPW, D), jnp.float32),
          pltpu.VMEM((BPW, D), jnp.float32),
          pltpu.VMEM((BPW,), jnp.float32),
          pltpu.VMEM((BPW,), jnp.float32),
          pltpu.VMEM((BPW,), jnp.float32),
          pltpu.SemaphoreType.DMA,
      ],
  )
  return f(uEmbeds, iEmbeds, uBias, iBias, usr, itm)


# speedup vs baseline: 1.0620x; 1.0620x over previous
import jax
import jax.numpy as jnp
from jax import lax
from jax.experimental import pallas as pl
from jax.experimental.pallas import tpu as pltpu
from jax.experimental.pallas import tpu_sc as plsc

B = 16384
D = 64
DP = 128
L = 16
NC = 2
NS = 16
NW = NC * NS
BPW = B // NW
CH = 2
BPC = BPW // CH
NB = BPC // L


def _bias_mf_body(u_hbm, i_hbm, ub_hbm, ib_hbm, usr_hbm, itm_hbm, out_hbm,
                  uidx, iidx, urows, irows, ubv, ibv, outv, bsem, dsem):
  wid = lax.axis_index("s") * NC + lax.axis_index("c")
  base = wid * BPW

  pltpu.sync_copy(usr_hbm.at[pl.ds(base, BPW)], uidx)
  pltpu.sync_copy(itm_hbm.at[pl.ds(base, BPW)], iidx)

  cub = pltpu.async_copy(ub_hbm.at[uidx], ubv, bsem)
  cib = pltpu.async_copy(ib_hbm.at[iidx], ibv, bsem)
  cub.wait()
  cib.wait()

  lane = lax.iota(jnp.int32, L)

  for h in range(CH):
    cu = pltpu.async_copy(u_hbm.at[uidx.at[pl.ds(h * BPC, BPC)]], urows, dsem)
    ci = pltpu.async_copy(i_hbm.at[iidx.at[pl.ds(h * BPC, BPC)]], irows, dsem)
    cu.wait()
    ci.wait()

    def batch(t, carry):
      rows = t * L + lane
      off = h * BPC + t * L
      acc = ubv[pl.ds(off, L)] + ibv[pl.ds(off, L)]
      for d in range(D):
        col = jnp.full((L,), d, jnp.int32)
        uv = plsc.load_gather(urows, [rows, col])
        iv = plsc.load_gather(irows, [rows, col])
        acc = acc + uv * iv
      outv[pl.ds(off, L)] = acc
      return carry

    lax.fori_loop(0, NB, batch, 0)

  pltpu.sync_copy(outv, out_hbm.at[pl.ds(base, BPW)])


def kernel(uEmbeds, iEmbeds, uBias, iBias, usr, itm):
  u2 = jnp.pad(uEmbeds, ((0, 0), (0, DP - D)))
  i2 = jnp.pad(iEmbeds, ((0, 0), (0, DP - D)))
  f = pl.kernel(
      _bias_mf_body,
      out_type=jax.ShapeDtypeStruct((B,), jnp.float32),
      mesh=plsc.VectorSubcoreMesh(core_axis_name="c", subcore_axis_name="s"),
      compiler_params=pltpu.CompilerParams(
          needs_layout_passes=False, use_tc_tiling_on_sc=True),
      scratch_types=[
          pltpu.VMEM((BPW,), jnp.int32),
          pltpu.VMEM((BPW,), jnp.int32),
          pltpu.VMEM((BPC, DP), jnp.float32),
          pltpu.VMEM((BPC, DP), jnp.float32),
          pltpu.VMEM((BPW,), jnp.float32),
          pltpu.VMEM((BPW,), jnp.float32),
          pltpu.VMEM((BPW,), jnp.float32),
          pltpu.SemaphoreType.DMA,
          pltpu.SemaphoreType.DMA,
      ],
  )
  return f(u2, i2, uBias, iBias, usr, itm)


# speedup vs baseline: 1.3415x; 1.2632x over previous
import jax
import jax.numpy as jnp
from jax import lax
from jax.experimental import pallas as pl
from jax.experimental.pallas import tpu as pltpu
from jax.experimental.pallas import tpu_sc as plsc

B = 16384
D = 64
DP = 128
V = 1000000
VH = V // 2
L = 16
NC = 2
NS = 16
NW = NC * NS
BPW = B // NW
CH = 2
BPC = BPW // CH
NB = BPC // L

COLS = 4096
GRID = -(-V // COLS)


def _relayout_body(ut_ref, it_ref, u2_ref, i2_ref):
  def merge(src):
    t = jnp.transpose(src[...], (1, 0))
    r = t.reshape(COLS // 2, 2, D)
    return jnp.concatenate([r[:, 0, :], r[:, 1, :]], axis=1)

  u2_ref[...] = merge(ut_ref)
  i2_ref[...] = merge(it_ref)


def _relayout(ut, it):
  return pl.pallas_call(
      _relayout_body,
      grid=(GRID,),
      in_specs=[
          pl.BlockSpec((D, COLS), lambda g: (0, g)),
          pl.BlockSpec((D, COLS), lambda g: (0, g)),
      ],
      out_specs=[
          pl.BlockSpec((COLS // 2, DP), lambda g: (g, 0)),
          pl.BlockSpec((COLS // 2, DP), lambda g: (g, 0)),
      ],
      out_shape=[
          jax.ShapeDtypeStruct((VH, DP), jnp.float32),
          jax.ShapeDtypeStruct((VH, DP), jnp.float32),
      ],
  )(ut, it)


def _bias_mf_body(u_hbm, i_hbm, ub_hbm, ib_hbm, usr_hbm, itm_hbm, out_hbm,
                  uidx, iidx, uhx, ihx, urows, irows, ubv, ibv, outv,
                  bsem, dsem):
  wid = lax.axis_index("s") * NC + lax.axis_index("c")
  base = wid * BPW

  pltpu.sync_copy(usr_hbm.at[pl.ds(base, BPW)], uidx)
  pltpu.sync_copy(itm_hbm.at[pl.ds(base, BPW)], iidx)

  cub = pltpu.async_copy(ub_hbm.at[uidx], ubv, bsem)
  cib = pltpu.async_copy(ib_hbm.at[iidx], ibv, bsem)

  def halve(t, carry):
    sl = pl.ds(t * L, L)
    uhx[sl] = lax.shift_right_logical(uidx[sl], 1)
    ihx[sl] = lax.shift_right_logical(iidx[sl], 1)
    return carry

  lax.fori_loop(0, BPW // L, halve, 0)

  cub.wait()
  cib.wait()

  lane = lax.iota(jnp.int32, L)

  for h in range(CH):
    cu = pltpu.async_copy(u_hbm.at[uhx.at[pl.ds(h * BPC, BPC)]], urows, dsem)
    ci = pltpu.async_copy(i_hbm.at[ihx.at[pl.ds(h * BPC, BPC)]], irows, dsem)
    cu.wait()
    ci.wait()

    def batch(t, carry):
      rows = t * L + lane
      off = h * BPC + t * L
      uoff = (uidx[pl.ds(off, L)] & 1) * D
      ioff = (iidx[pl.ds(off, L)] & 1) * D
      acc = ubv[pl.ds(off, L)] + ibv[pl.ds(off, L)]
      for d in range(D):
        uv = plsc.load_gather(urows, [rows, uoff + d])
        iv = plsc.load_gather(irows, [rows, ioff + d])
        acc = acc + uv * iv
      outv[pl.ds(off, L)] = acc
      return carry

    lax.fori_loop(0, NB, batch, 0)

  pltpu.sync_copy(outv, out_hbm.at[pl.ds(base, BPW)])


def kernel(uEmbeds, iEmbeds, uBias, iBias, usr, itm):
  u2, i2 = _relayout(uEmbeds.T, iEmbeds.T)
  f = pl.kernel(
      _bias_mf_body,
      out_type=jax.ShapeDtypeStruct((B,), jnp.float32),
      mesh=plsc.VectorSubcoreMesh(core_axis_name="c", subcore_axis_name="s"),
      compiler_params=pltpu.CompilerParams(
          needs_layout_passes=False, use_tc_tiling_on_sc=True),
      scratch_types=[
          pltpu.VMEM((BPW,), jnp.int32),
          pltpu.VMEM((BPW,), jnp.int32),
          pltpu.VMEM((BPW,), jnp.int32),
          pltpu.VMEM((BPW,), jnp.int32),
          pltpu.VMEM((BPC, DP), jnp.float32),
          pltpu.VMEM((BPC, DP), jnp.float32),
          pltpu.VMEM((BPW,), jnp.float32),
          pltpu.VMEM((BPW,), jnp.float32),
          pltpu.VMEM((BPW,), jnp.float32),
          pltpu.SemaphoreType.DMA,
          pltpu.SemaphoreType.DMA,
      ],
  )
  return f(u2, i2, uBias, iBias, usr, itm)


# speedup vs baseline: 2.0126x; 1.5003x over previous
import jax
import jax.numpy as jnp
from jax import lax
from jax.experimental import pallas as pl
from jax.experimental.pallas import tpu as pltpu
from jax.experimental.pallas import tpu_sc as plsc

B = 16384
D = 64
DP = 128
V = 1000000
L = 16
NC = 2
NS = 16
NW = NC * NS
BPW = B // NW
CH = 2
BPC = BPW // CH
NB = BPC // L

COLS = 4096
GRID = -(-V // COLS)


def _relayout_body(ut_ref, it_ref, o_ref):
  o_ref[:, 0:D] = jnp.transpose(ut_ref[...], (1, 0))
  o_ref[:, D:DP] = jnp.transpose(it_ref[...], (1, 0))


def _relayout(ut, it):
  return pl.pallas_call(
      _relayout_body,
      grid=(GRID,),
      in_specs=[
          pl.BlockSpec((D, COLS), lambda g: (0, g)),
          pl.BlockSpec((D, COLS), lambda g: (0, g)),
      ],
      out_specs=pl.BlockSpec((COLS, DP), lambda g: (g, 0)),
      out_shape=jax.ShapeDtypeStruct((V, DP), jnp.float32),
  )(ut, it)


def _bias_mf_body(tab_hbm, ub_hbm, ib_hbm, usr_hbm, itm_hbm, out_hbm,
                  uidx, iidx, urows, irows, ubv, ibv, outv, bsem, dsem):
  wid = lax.axis_index("s") * NC + lax.axis_index("c")
  base = wid * BPW

  pltpu.sync_copy(usr_hbm.at[pl.ds(base, BPW)], uidx)
  pltpu.sync_copy(itm_hbm.at[pl.ds(base, BPW)], iidx)

  cub = pltpu.async_copy(ub_hbm.at[uidx], ubv, bsem)
  cib = pltpu.async_copy(ib_hbm.at[iidx], ibv, bsem)
  cub.wait()
  cib.wait()

  lane = lax.iota(jnp.int32, L)

  for h in range(CH):
    cu = pltpu.async_copy(tab_hbm.at[uidx.at[pl.ds(h * BPC, BPC)]], urows, dsem)
    ci = pltpu.async_copy(tab_hbm.at[iidx.at[pl.ds(h * BPC, BPC)]], irows, dsem)
    cu.wait()
    ci.wait()

    def batch(t, carry):
      rows = t * L + lane
      off = h * BPC + t * L
      acc = ubv[pl.ds(off, L)] + ibv[pl.ds(off, L)]
      for d in range(D):
        uv = plsc.load_gather(urows, [rows, jnp.full((L,), d, jnp.int32)])
        iv = plsc.load_gather(irows, [rows, jnp.full((L,), D + d, jnp.int32)])
        acc = acc + uv * iv
      outv[pl.ds(off, L)] = acc
      return carry

    lax.fori_loop(0, NB, batch, 0)

  pltpu.sync_copy(outv, out_hbm.at[pl.ds(base, BPW)])


def kernel(uEmbeds, iEmbeds, uBias, iBias, usr, itm):
  tab = _relayout(uEmbeds.T, iEmbeds.T)
  f = pl.kernel(
      _bias_mf_body,
      out_type=jax.ShapeDtypeStruct((B,), jnp.float32),
      mesh=plsc.VectorSubcoreMesh(core_axis_name="c", subcore_axis_name="s"),
      compiler_params=pltpu.CompilerParams(
          needs_layout_passes=False, use_tc_tiling_on_sc=True),
      scratch_types=[
          pltpu.VMEM((BPW,), jnp.int32),
          pltpu.VMEM((BPW,), jnp.int32),
          pltpu.VMEM((BPC, DP), jnp.float32),
          pltpu.VMEM((BPC, DP), jnp.float32),
          pltpu.VMEM((BPW,), jnp.float32),
          pltpu.VMEM((BPW,), jnp.float32),
          pltpu.VMEM((BPW,), jnp.float32),
          pltpu.SemaphoreType.DMA,
          pltpu.SemaphoreType.DMA,
      ],
  )
  return f(tab, uBias, iBias, usr, itm)


# speedup vs baseline: 2.5671x; 1.2755x over previous
import jax
import jax.numpy as jnp
from jax import lax
from jax.experimental import pallas as pl
from jax.experimental.pallas import tpu as pltpu
from jax.experimental.pallas import tpu_sc as plsc

B = 16384
D = 64
DP = 128
V = 1000000
L = 16
NC = 2
NS = 16
NW = NC * NS
BPW = B // NW
CH = 2
BPC = BPW // CH
NB = BPC // L

COLS = 4096
GRID = -(-V // COLS)


def _relayout_body(ut_ref, it_ref, o_ref):
  both = jnp.concatenate([ut_ref[...], it_ref[...]], axis=0)
  o_ref[...] = jnp.transpose(both, (1, 0))


def _relayout(ut, it):
  return pl.pallas_call(
      _relayout_body,
      grid=(GRID,),
      in_specs=[
          pl.BlockSpec((D, COLS), lambda g: (0, g)),
          pl.BlockSpec((D, COLS), lambda g: (0, g)),
      ],
      out_specs=pl.BlockSpec((COLS, DP), lambda g: (g, 0)),
      out_shape=jax.ShapeDtypeStruct((V, DP), jnp.float32),
  )(ut, it)


def _bias_mf_body(tab_hbm, ub_hbm, ib_hbm, usr_hbm, itm_hbm, out_hbm,
                  uidx, iidx, urows, irows, ubv, ibv, outv, bsem, dsem):
  wid = lax.axis_index("s") * NC + lax.axis_index("c")
  base = wid * BPW

  pltpu.sync_copy(usr_hbm.at[pl.ds(base, BPW)], uidx)
  pltpu.sync_copy(itm_hbm.at[pl.ds(base, BPW)], iidx)

  cub = pltpu.async_copy(ub_hbm.at[uidx], ubv, bsem)
  cib = pltpu.async_copy(ib_hbm.at[iidx], ibv, bsem)
  cub.wait()
  cib.wait()

  lane = lax.iota(jnp.int32, L)

  for h in range(CH):
    cu = pltpu.async_copy(tab_hbm.at[uidx.at[pl.ds(h * BPC, BPC)]], urows, dsem)
    ci = pltpu.async_copy(tab_hbm.at[iidx.at[pl.ds(h * BPC, BPC)]], irows, dsem)
    cu.wait()
    ci.wait()

    def batch(t, carry):
      rows = t * L + lane
      off = h * BPC + t * L
      acc = ubv[pl.ds(off, L)] + ibv[pl.ds(off, L)]
      for d in range(D):
        uv = plsc.load_gather(urows, [rows, jnp.full((L,), d, jnp.int32)])
        iv = plsc.load_gather(irows, [rows, jnp.full((L,), D + d, jnp.int32)])
        acc = acc + uv * iv
      outv[pl.ds(off, L)] = acc
      return carry

    lax.fori_loop(0, NB, batch, 0)

  pltpu.sync_copy(outv, out_hbm.at[pl.ds(base, BPW)])


def kernel(uEmbeds, iEmbeds, uBias, iBias, usr, itm):
  tab = _relayout(uEmbeds.T, iEmbeds.T)
  f = pl.kernel(
      _bias_mf_body,
      out_type=jax.ShapeDtypeStruct((B,), jnp.float32),
      mesh=plsc.VectorSubcoreMesh(core_axis_name="c", subcore_axis_name="s"),
      compiler_params=pltpu.CompilerParams(
          needs_layout_passes=False, use_tc_tiling_on_sc=True),
      scratch_types=[
          pltpu.VMEM((BPW,), jnp.int32),
          pltpu.VMEM((BPW,), jnp.int32),
          pltpu.VMEM((BPC, DP), jnp.float32),
          pltpu.VMEM((BPC, DP), jnp.float32),
          pltpu.VMEM((BPW,), jnp.float32),
          pltpu.VMEM((BPW,), jnp.float32),
          pltpu.VMEM((BPW,), jnp.float32),
          pltpu.SemaphoreType.DMA,
          pltpu.SemaphoreType.DMA,
      ],
  )
  return f(tab, uBias, iBias, usr, itm)


# speedup vs baseline: 2.9551x; 1.1511x over previous
import jax
import jax.numpy as jnp
from jax import lax
from jax.experimental import pallas as pl
from jax.experimental.pallas import tpu as pltpu
from jax.experimental.pallas import tpu_sc as plsc

B = 16384
D = 64
DP = 128
V = 1000000
L = 16
NC = 2
NS = 16
NW = NC * NS
BPW = B // NW
CH = 2
BPC = BPW // CH
NB = BPC // L

COLS = 8192
GRID = -(-V // COLS)


def _relayout_body(ut_ref, it_ref, o_ref):
  both = jnp.concatenate([ut_ref[...], it_ref[...]], axis=0)
  o_ref[...] = jnp.transpose(both, (1, 0))


def _relayout(ut, it):
  return pl.pallas_call(
      _relayout_body,
      grid=(GRID,),
      in_specs=[
          pl.BlockSpec((D, COLS), lambda g: (0, g)),
          pl.BlockSpec((D, COLS), lambda g: (0, g)),
      ],
      out_specs=pl.BlockSpec((COLS, DP), lambda g: (g, 0)),
      out_shape=jax.ShapeDtypeStruct((V, DP), jnp.float32),
  )(ut, it)


def _bias_mf_body(tab_hbm, ub_hbm, ib_hbm, usr_hbm, itm_hbm, out_hbm,
                  uidx, iidx, urows, irows, ubv, ibv, outv, bsem, dsem):
  wid = lax.axis_index("s") * NC + lax.axis_index("c")
  base = wid * BPW

  pltpu.sync_copy(usr_hbm.at[pl.ds(base, BPW)], uidx)
  pltpu.sync_copy(itm_hbm.at[pl.ds(base, BPW)], iidx)

  cub = pltpu.async_copy(ub_hbm.at[uidx], ubv, bsem)
  cib = pltpu.async_copy(ib_hbm.at[iidx], ibv, bsem)
  cub.wait()
  cib.wait()

  lane = lax.iota(jnp.int32, L)

  for h in range(CH):
    cu = pltpu.async_copy(tab_hbm.at[uidx.at[pl.ds(h * BPC, BPC)]], urows, dsem)
    ci = pltpu.async_copy(tab_hbm.at[iidx.at[pl.ds(h * BPC, BPC)]], irows, dsem)
    cu.wait()
    ci.wait()

    def batch(t, carry):
      rows = t * L + lane
      off = h * BPC + t * L
      acc = ubv[pl.ds(off, L)] + ibv[pl.ds(off, L)]
      for d in range(D):
        uv = plsc.load_gather(urows, [rows, jnp.full((L,), d, jnp.int32)])
        iv = plsc.load_gather(irows, [rows, jnp.full((L,), D + d, jnp.int32)])
        acc = acc + uv * iv
      outv[pl.ds(off, L)] = acc
      return carry

    lax.fori_loop(0, NB, batch, 0)

  pltpu.sync_copy(outv, out_hbm.at[pl.ds(base, BPW)])


def kernel(uEmbeds, iEmbeds, uBias, iBias, usr, itm):
  tab = _relayout(uEmbeds.T, iEmbeds.T)
  f = pl.kernel(
      _bias_mf_body,
      out_type=jax.ShapeDtypeStruct((B,), jnp.float32),
      mesh=plsc.VectorSubcoreMesh(core_axis_name="c", subcore_axis_name="s"),
      compiler_params=pltpu.CompilerParams(
          needs_layout_passes=False, use_tc_tiling_on_sc=True),
      scratch_types=[
          pltpu.VMEM((BPW,), jnp.int32),
          pltpu.VMEM((BPW,), jnp.int32),
          pltpu.VMEM((BPC, DP), jnp.float32),
          pltpu.VMEM((BPC, DP), jnp.float32),
          pltpu.VMEM((BPW,), jnp.float32),
          pltpu.VMEM((BPW,), jnp.float32),
          pltpu.VMEM((BPW,), jnp.float32),
          pltpu.SemaphoreType.DMA,
          pltpu.SemaphoreType.DMA,
      ],
  )
  return f(tab, uBias, iBias, usr, itm)


# speedup vs baseline: 3.0265x; 1.0242x over previous
import jax
import jax.numpy as jnp
from jax import lax
from jax.experimental import pallas as pl
from jax.experimental.pallas import tpu as pltpu
from jax.experimental.pallas import tpu_sc as plsc

B = 16384
D = 64
DP = 128
V = 1000000
L = 16
NC = 2
NS = 16
NW = NC * NS
BPW = B // NW
CH = 2
BPC = BPW // CH
NB = BPC // L

COLS = 16384
GRID = -(-V // COLS)


def _relayout_body(ut_ref, it_ref, o_ref):
  both = jnp.concatenate([ut_ref[...], it_ref[...]], axis=0)
  o_ref[...] = jnp.transpose(both, (1, 0))


def _relayout(ut, it):
  return pl.pallas_call(
      _relayout_body,
      grid=(GRID,),
      in_specs=[
          pl.BlockSpec((D, COLS), lambda g: (0, g)),
          pl.BlockSpec((D, COLS), lambda g: (0, g)),
      ],
      out_specs=pl.BlockSpec((COLS, DP), lambda g: (g, 0)),
      out_shape=jax.ShapeDtypeStruct((V, DP), jnp.float32),
  )(ut, it)


def _bias_mf_body(tab_hbm, ub_hbm, ib_hbm, usr_hbm, itm_hbm, out_hbm,
                  uidx, iidx, urows, irows, ubv, ibv, outv, bsem, dsem):
  wid = lax.axis_index("s") * NC + lax.axis_index("c")
  base = wid * BPW

  pltpu.sync_copy(usr_hbm.at[pl.ds(base, BPW)], uidx)
  pltpu.sync_copy(itm_hbm.at[pl.ds(base, BPW)], iidx)

  cub = pltpu.async_copy(ub_hbm.at[uidx], ubv, bsem)
  cib = pltpu.async_copy(ib_hbm.at[iidx], ibv, bsem)
  cub.wait()
  cib.wait()

  lane = lax.iota(jnp.int32, L)

  for h in range(CH):
    cu = pltpu.async_copy(tab_hbm.at[uidx.at[pl.ds(h * BPC, BPC)]], urows, dsem)
    ci = pltpu.async_copy(tab_hbm.at[iidx.at[pl.ds(h * BPC, BPC)]], irows, dsem)
    cu.wait()
    ci.wait()

    def batch(t, carry):
      rows = t * L + lane
      off = h * BPC + t * L
      acc = ubv[pl.ds(off, L)] + ibv[pl.ds(off, L)]
      for d in range(D):
        uv = plsc.load_gather(urows, [rows, jnp.full((L,), d, jnp.int32)])
        iv = plsc.load_gather(irows, [rows, jnp.full((L,), D + d, jnp.int32)])
        acc = acc + uv * iv
      outv[pl.ds(off, L)] = acc
      return carry

    lax.fori_loop(0, NB, batch, 0)

  pltpu.sync_copy(outv, out_hbm.at[pl.ds(base, BPW)])


def kernel(uEmbeds, iEmbeds, uBias, iBias, usr, itm):
  tab = _relayout(uEmbeds.T, iEmbeds.T)
  f = pl.kernel(
      _bias_mf_body,
      out_type=jax.ShapeDtypeStruct((B,), jnp.float32),
      mesh=plsc.VectorSubcoreMesh(core_axis_name="c", subcore_axis_name="s"),
      compiler_params=pltpu.CompilerParams(
          needs_layout_passes=False, use_tc_tiling_on_sc=True),
      scratch_types=[
          pltpu.VMEM((BPW,), jnp.int32),
          pltpu.VMEM((BPW,), jnp.int32),
          pltpu.VMEM((BPC, DP), jnp.float32),
          pltpu.VMEM((BPC, DP), jnp.float32),
          pltpu.VMEM((BPW,), jnp.float32),
          pltpu.VMEM((BPW,), jnp.float32),
          pltpu.VMEM((BPW,), jnp.float32),
          pltpu.SemaphoreType.DMA,
          pltpu.SemaphoreType.DMA,
      ],
  )
  return f(tab, uBias, iBias, usr, itm)


# speedup vs baseline: 3.0336x; 1.0023x over previous
import jax
import jax.numpy as jnp
from jax import lax
from jax.experimental import pallas as pl
from jax.experimental.pallas import tpu as pltpu
from jax.experimental.pallas import tpu_sc as plsc

B = 16384
D = 64
DP = 128
V = 1000000
L = 16
NC = 2
NS = 16
NW = NC * NS
BPW = B // NW
CH = 2
BPC = BPW // CH
NB = BPC // L

COLS = 24576
GRID = -(-V // COLS)


def _relayout_body(ut_ref, it_ref, o_ref):
  both = jnp.concatenate([ut_ref[...], it_ref[...]], axis=0)
  o_ref[...] = jnp.transpose(both, (1, 0))


def _relayout(ut, it):
  return pl.pallas_call(
      _relayout_body,
      grid=(GRID,),
      in_specs=[
          pl.BlockSpec((D, COLS), lambda g: (0, g)),
          pl.BlockSpec((D, COLS), lambda g: (0, g)),
      ],
      out_specs=pl.BlockSpec((COLS, DP), lambda g: (g, 0)),
      out_shape=jax.ShapeDtypeStruct((V, DP), jnp.float32),
  )(ut, it)


def _bias_mf_body(tab_hbm, ub_hbm, ib_hbm, usr_hbm, itm_hbm, out_hbm,
                  uidx, iidx, urows, irows, ubv, ibv, outv, bsem, dsem):
  wid = lax.axis_index("s") * NC + lax.axis_index("c")
  base = wid * BPW

  pltpu.sync_copy(usr_hbm.at[pl.ds(base, BPW)], uidx)
  pltpu.sync_copy(itm_hbm.at[pl.ds(base, BPW)], iidx)

  cub = pltpu.async_copy(ub_hbm.at[uidx], ubv, bsem)
  cib = pltpu.async_copy(ib_hbm.at[iidx], ibv, bsem)
  cub.wait()
  cib.wait()

  lane = lax.iota(jnp.int32, L)

  for h in range(CH):
    cu = pltpu.async_copy(tab_hbm.at[uidx.at[pl.ds(h * BPC, BPC)]], urows, dsem)
    ci = pltpu.async_copy(tab_hbm.at[iidx.at[pl.ds(h * BPC, BPC)]], irows, dsem)
    cu.wait()
    ci.wait()

    def batch(t, carry):
      rows = t * L + lane
      off = h * BPC + t * L
      acc = ubv[pl.ds(off, L)] + ibv[pl.ds(off, L)]
      for d in range(D):
        uv = plsc.load_gather(urows, [rows, jnp.full((L,), d, jnp.int32)])
        iv = plsc.load_gather(irows, [rows, jnp.full((L,), D + d, jnp.int32)])
        acc = acc + uv * iv
      outv[pl.ds(off, L)] = acc
      return carry

    lax.fori_loop(0, NB, batch, 0)

  pltpu.sync_copy(outv, out_hbm.at[pl.ds(base, BPW)])


def kernel(uEmbeds, iEmbeds, uBias, iBias, usr, itm):
  tab = _relayout(uEmbeds.T, iEmbeds.T)
  f = pl.kernel(
      _bias_mf_body,
      out_type=jax.ShapeDtypeStruct((B,), jnp.float32),
      mesh=plsc.VectorSubcoreMesh(core_axis_name="c", subcore_axis_name="s"),
      compiler_params=pltpu.CompilerParams(
          needs_layout_passes=False, use_tc_tiling_on_sc=True),
      scratch_types=[
          pltpu.VMEM((BPW,), jnp.int32),
          pltpu.VMEM((BPW,), jnp.int32),
          pltpu.VMEM((BPC, DP), jnp.float32),
          pltpu.VMEM((BPC, DP), jnp.float32),
          pltpu.VMEM((BPW,), jnp.float32),
          pltpu.VMEM((BPW,), jnp.float32),
          pltpu.VMEM((BPW,), jnp.float32),
          pltpu.SemaphoreType.DMA,
          pltpu.SemaphoreType.DMA,
      ],
  )
  return f(tab, uBias, iBias, usr, itm)


# speedup vs baseline: 3.0773x; 1.0144x over previous
import jax
import jax.numpy as jnp
from jax import lax
from jax.experimental import pallas as pl
from jax.experimental.pallas import tpu as pltpu
from jax.experimental.pallas import tpu_sc as plsc

B = 16384
D = 64
DP = 128
V = 1000000
L = 16
NC = 2
NS = 16
NW = NC * NS
BPW = B // NW
CH = 4
BPC = BPW // CH
NB = BPC // L

COLS = 24576
GRID = -(-V // COLS)


def _relayout_body(ut_ref, it_ref, o_ref):
  both = jnp.concatenate([ut_ref[...], it_ref[...]], axis=0)
  o_ref[...] = jnp.transpose(both, (1, 0))


def _relayout(ut, it):
  return pl.pallas_call(
      _relayout_body,
      grid=(GRID,),
      in_specs=[
          pl.BlockSpec((D, COLS), lambda g: (0, g)),
          pl.BlockSpec((D, COLS), lambda g: (0, g)),
      ],
      out_specs=pl.BlockSpec((COLS, DP), lambda g: (g, 0)),
      out_shape=jax.ShapeDtypeStruct((V, DP), jnp.float32),
  )(ut, it)


def _bias_mf_body(tab_hbm, ub_hbm, ib_hbm, usr_hbm, itm_hbm, out_hbm,
                  uidx, iidx, ur0, ir0, ur1, ir1, ubv, ibv, outv,
                  bsem, dsem0, dsem1):
  wid = lax.axis_index("s") * NC + lax.axis_index("c")
  base = wid * BPW

  pltpu.sync_copy(usr_hbm.at[pl.ds(base, BPW)], uidx)
  pltpu.sync_copy(itm_hbm.at[pl.ds(base, BPW)], iidx)

  cub = pltpu.async_copy(ub_hbm.at[uidx], ubv, bsem)
  cib = pltpu.async_copy(ib_hbm.at[iidx], ibv, bsem)

  ubufs = (ur0, ur1)
  ibufs = (ir0, ir1)
  sems = (dsem0, dsem1)

  def fetch(h):
    p = h % 2
    cu = pltpu.async_copy(
        tab_hbm.at[uidx.at[pl.ds(h * BPC, BPC)]], ubufs[p], sems[p])
    ci = pltpu.async_copy(
        tab_hbm.at[iidx.at[pl.ds(h * BPC, BPC)]], ibufs[p], sems[p])
    return cu, ci

  pend = fetch(0)
  cub.wait()
  cib.wait()

  lane = lax.iota(jnp.int32, L)

  for h in range(CH):
    cu, ci = pend
    if h + 1 < CH:
      nxt = fetch(h + 1)
    cu.wait()
    ci.wait()
    if h + 1 < CH:
      pend = nxt
    p = h % 2
    urows = ubufs[p]
    irows = ibufs[p]

    def batch(t, carry):
      rows = t * L + lane
      off = h * BPC + t * L
      acc = ubv[pl.ds(off, L)] + ibv[pl.ds(off, L)]
      for d in range(D):
        uv = plsc.load_gather(urows, [rows, jnp.full((L,), d, jnp.int32)])
        iv = plsc.load_gather(irows, [rows, jnp.full((L,), D + d, jnp.int32)])
        acc = acc + uv * iv
      outv[pl.ds(off, L)] = acc
      return carry

    lax.fori_loop(0, NB, batch, 0)

  pltpu.sync_copy(outv, out_hbm.at[pl.ds(base, BPW)])


def kernel(uEmbeds, iEmbeds, uBias, iBias, usr, itm):
  tab = _relayout(uEmbeds.T, iEmbeds.T)
  f = pl.kernel(
      _bias_mf_body,
      out_type=jax.ShapeDtypeStruct((B,), jnp.float32),
      mesh=plsc.VectorSubcoreMesh(core_axis_name="c", subcore_axis_name="s"),
      compiler_params=pltpu.CompilerParams(
          needs_layout_passes=False, use_tc_tiling_on_sc=True),
      scratch_types=[
          pltpu.VMEM((BPW,), jnp.int32),
          pltpu.VMEM((BPW,), jnp.int32),
          pltpu.VMEM((BPC, DP), jnp.float32),
          pltpu.VMEM((BPC, DP), jnp.float32),
          pltpu.VMEM((BPC, DP), jnp.float32),
          pltpu.VMEM((BPC, DP), jnp.float32),
          pltpu.VMEM((BPW,), jnp.float32),
          pltpu.VMEM((BPW,), jnp.float32),
          pltpu.VMEM((BPW,), jnp.float32),
          pltpu.SemaphoreType.DMA,
          pltpu.SemaphoreType.DMA,
          pltpu.SemaphoreType.DMA,
      ],
  )
  return f(tab, uBias, iBias, usr, itm)
